# Initial kernel scaffold; baseline (speedup 1.0000x reference)
import jax
import jax.numpy as jnp
from jax.experimental import pallas as pl


def kernel(x, edge_index, batch, W1, as1, ad1, b1, W2, as2, ad2, b2, W3, as3, ad3, b3, W4, as4, ad4, b4):
    raise NotImplementedError("write your pallas kernel here")


# speedup vs baseline: 15.1398x; 15.1398x over previous
import functools

import jax
import jax.numpy as jnp
from jax import lax
from jax.experimental import pallas as pl
from jax.experimental.pallas import tpu as pltpu
from jax.experimental.pallas import tpu_sc as plsc

N = 10000
E = 160000
G = 64
H = 320
HH = H // 2
ETOT = E + N
NS = 16
NC = 2
K = 64
SB = 8
CHUNKS = SB * (-(-ETOT // (NS * K * SB)))
EPT = CHUNKS * K
EPAD = EPT * NS
BR = 400
NBLK = N // BR
RAL = 624
RCNT = 640


def _prep_body(first, residual, *refs):
    if first:
        if residual:
            raise AssertionError
        (x_ref, w_ref, as_ref, ad_ref,
         h_ref, es_ref, ed_ref, mx_ref) = refs
        f = x_ref[...]
    else:
        if residual:
            (alo_ref, ahi_ref, den_ref, b_ref, res_ref, w_ref, as_ref,
             ad_ref,
             out_ref, h_ref, es_ref, ed_ref, mx_ref) = refs
        else:
            (alo_ref, ahi_ref, den_ref, b_ref, w_ref, as_ref, ad_ref,
             out_ref, h_ref, es_ref, ed_ref, mx_ref) = refs
        acc = jnp.concatenate([alo_ref[...], ahi_ref[...]], axis=1)
        den = jnp.maximum(den_ref[...], 1e-30)
        outp = jnp.maximum(acc / den + b_ref[...], 0.0)
        out_ref[...] = outp
        f = outp + res_ref[...] if residual else outp

    w = w_ref[...]
    h_ref[...] = jnp.dot(f, w, preferred_element_type=jnp.float32)
    was = jnp.dot(w, as_ref[0, :], preferred_element_type=jnp.float32)
    wad = jnp.dot(w, ad_ref[0, :], preferred_element_type=jnp.float32)
    es = jnp.dot(f, was[:, None], preferred_element_type=jnp.float32)
    ed = jnp.dot(f, wad[:, None], preferred_element_type=jnp.float32)
    es_ref[...] = es
    ed_ref[...] = ed
    row = jnp.concatenate([jnp.max(es).reshape(1), jnp.max(ed).reshape(1),
                           jnp.zeros((6,), jnp.float32)])
    mx_ref[...] = row.reshape(1, 1, 8)


def _tc_prep(first, residual, d_in, args):
    whole = lambda shape: pl.BlockSpec(shape, lambda i: (0,) * len(shape))
    rows = lambda w: pl.BlockSpec((BR, w), lambda i: (i, 0))
    in_specs = []
    if first:
        in_specs.append(rows(3))
    else:
        in_specs.append(pl.BlockSpec((BR, HH), lambda i: (i, 0)))
        in_specs.append(pl.BlockSpec((BR, HH), lambda i: (NBLK + i, 0)))
        in_specs.append(rows(1))
        in_specs.append(whole((1, H)))
        if residual:
            in_specs.append(rows(H))
    in_specs += [whole((d_in, H)), whole((1, H)), whole((1, H))]
    out_shapes = []
    out_specs = []
    if not first:
        out_shapes.append(jax.ShapeDtypeStruct((N, H), jnp.float32))
        out_specs.append(rows(H))
    out_shapes += [jax.ShapeDtypeStruct((N, H), jnp.float32),
                   jax.ShapeDtypeStruct((N, 1), jnp.float32),
                   jax.ShapeDtypeStruct((N, 1), jnp.float32),
                   jax.ShapeDtypeStruct((NBLK, 1, 8), jnp.float32)]
    out_specs += [rows(H), rows(1), rows(1),
                  pl.BlockSpec((1, 1, 8), lambda i: (i, 0, 0))]
    return pl.pallas_call(
        functools.partial(_prep_body, first, residual),
        grid=(NBLK,),
        in_specs=in_specs,
        out_specs=out_specs,
        out_shape=out_shapes,
    )(*args)


def _sc_edge_body(hcat, esrc, edst, srcp, dstp, mvec,
                  acc_out, den_out,
                  acc_sp, den_sp, esrc_sp, edst_sp,
                  sbuf_v, dbuf_v,
                  isrc_v, rsrc_v, idst_v, esb_v, edb_v, ex_v, rows_v,
                  zrow_v, m_v,
                  gsem, ssem, esem, edsem, dsem):
    c = lax.axis_index("c")
    s = lax.axis_index("s")

    def zrow_body(i, _):
        zrow_v[pl.ds(i * 16, 16)] = jnp.zeros((16,), jnp.float32)
        return 0
    lax.fori_loop(0, RCNT // 16, zrow_body, 0)

    def zr_body(jj, _):
        def cg_body(cg, _):
            rows_v[0, jj, pl.ds(cg * 16, 16)] = jnp.zeros((16,), jnp.float32)
            return 0
        lax.fori_loop(0, HH // 16, cg_body, 0)
        return 0
    lax.fori_loop(0, K, zr_body, 0)

    r0 = s * RAL
    for blk in range(RCNT // K):
        pltpu.async_copy(rows_v.at[0], acc_sp.at[pl.ds(r0 + blk * K, K), :],
                         gsem.at[0])
    pltpu.sync_copy(zrow_v, den_sp.at[pl.ds(r0, RCNT)])
    for blk in range(RCNT // K):
        pltpu.make_async_copy(rows_v.at[0],
                              acc_sp.at[pl.ds(r0 + blk * K, K), :],
                              gsem.at[0]).wait()

    pltpu.sync_copy(esrc.at[pl.ds(r0, RCNT)], zrow_v)
    pltpu.sync_copy(zrow_v, esrc_sp.at[pl.ds(r0, RCNT)])
    pltpu.sync_copy(edst.at[pl.ds(r0, RCNT)], zrow_v)
    pltpu.sync_copy(zrow_v, edst_sp.at[pl.ds(r0, RCNT)])
    pltpu.sync_copy(mvec, m_v)

    plsc.subcore_barrier()

    tile_base = s * EPT

    def build_idx(ch, slot):
        @pl.when(ch % SB == 0)
        def _():
            hb = tile_base + ch * K
            pltpu.sync_copy(srcp.at[pl.ds(hb, SB * K)], sbuf_v)
            pltpu.sync_copy(dstp.at[pl.ds(hb, SB * K)], dbuf_v)

        off = (ch % SB) * K
        for g in range(K // 16):
            s16 = sbuf_v[pl.ds(off + g * 16, 16)]
            d16 = dbuf_v[pl.ds(off + g * 16, 16)]
            isrc_v[slot, pl.ds(g * 16, 16)] = 2 * s16 + c
            rsrc_v[slot, pl.ds(g * 16, 16)] = s16
            idst_v[slot, pl.ds(g * 16, 16)] = d16

    def gather_rows(slot):
        pltpu.async_copy(hcat.at[isrc_v.at[slot]], rows_v.at[slot],
                         gsem.at[slot])
        pltpu.async_copy(esrc_sp.at[rsrc_v.at[slot]], esb_v.at[slot],
                         esem.at[slot])
        pltpu.async_copy(edst_sp.at[idst_v.at[slot]], edb_v.at[slot],
                         edsem.at[slot])

    def compute_ex(ch, slot):
        mv = m_v[...]
        lanes = lax.iota(jnp.int32, 16)
        pltpu.make_async_copy(esrc_sp.at[rsrc_v.at[slot]], esb_v.at[slot],
                              esem.at[slot]).wait()
        pltpu.make_async_copy(edst_sp.at[idst_v.at[slot]], edb_v.at[slot],
                              edsem.at[slot]).wait()
        for g in range(K // 16):
            es = esb_v[slot, pl.ds(g * 16, 16)]
            ed = edb_v[slot, pl.ds(g * 16, 16)]
            z = es + ed
            l = jnp.where(z >= 0.0, z, 0.2 * z)
            exv = jnp.exp(l - mv)
            eg = tile_base + ch * K + g * 16 + lanes
            exv = jnp.where(eg < ETOT, exv, 0.0)
            ex_v[slot, pl.ds(g * 16, 16)] = exv

        @pl.when(c == 0)
        def _():
            pltpu.async_copy(ex_v.at[slot], den_sp.at[idst_v.at[slot]],
                             dsem.at[slot], add=True)

    def weight_rows(slot):
        @plsc.parallel_loop(0, K // 16)
        def wrow_body(jg):
            exg = ex_v[slot, pl.ds(jg * 16, 16)]
            for jj in range(16):
                j = jg * 16 + jj
                exj = exg[jj]
                for cg in range(HH // 16):
                    rows_v[slot, j, pl.ds(cg * 16, 16)] = (
                        rows_v[slot, j, pl.ds(cg * 16, 16)] * exj)

    def scatter_rows(slot, ssem):
        return pltpu.async_copy(rows_v.at[slot], acc_sp.at[idst_v.at[slot]],
                                ssem, add=True)

    def process(ch, slot):
        other = 1 - slot

        @pl.when(ch + 1 < CHUNKS)
        def _():
            @pl.when(ch >= 1)
            def _():
                pltpu.make_async_copy(rows_v.at[other],
                                      acc_sp.at[idst_v.at[other]],
                                      ssem.at[other]).wait()

                @pl.when(c == 0)
                def _():
                    pltpu.make_async_copy(ex_v.at[other],
                                          den_sp.at[idst_v.at[other]],
                                          dsem.at[other]).wait()
            build_idx(ch + 1, other)
            gather_rows(other)

        compute_ex(ch, slot)
        pltpu.make_async_copy(hcat.at[isrc_v.at[slot]], rows_v.at[slot],
                              gsem.at[slot]).wait()
        weight_rows(slot)
        scatter_rows(slot, ssem.at[slot])

    build_idx(0, 0)
    gather_rows(0)

    def pair_body(p, _):
        process(2 * p, 0)
        process(2 * p + 1, 1)
        return 0

    lax.fori_loop(0, CHUNKS // 2, pair_body, 0)
    for sl in range(2):
        pltpu.make_async_copy(rows_v.at[sl], acc_sp.at[idst_v.at[sl]],
                              ssem.at[sl]).wait()

        @pl.when(c == 0)
        def _():
            pltpu.make_async_copy(ex_v.at[sl], den_sp.at[idst_v.at[sl]],
                                  dsem.at[sl]).wait()

    plsc.subcore_barrier()

    for blk in range(RCNT // K):
        pltpu.async_copy(acc_sp.at[pl.ds(r0 + blk * K, K), :],
                         acc_out.at[pl.ds(c * N + r0 + blk * K, K), :],
                         ssem.at[0])
    for blk in range(RCNT // K):
        pltpu.make_async_copy(acc_sp.at[pl.ds(r0 + blk * K, K), :],
                              acc_out.at[pl.ds(c * N + r0 + blk * K, K), :],
                              ssem.at[0]).wait()

    @pl.when(c == 0)
    def _():
        pltpu.sync_copy(den_sp.at[pl.ds(r0, RCNT)], den_out.at[pl.ds(r0, RCNT)])


@functools.lru_cache(maxsize=1)
def _get_sc_edge():
  return pl.kernel(
    _sc_edge_body,
    out_type=[jax.ShapeDtypeStruct((2 * N, HH), jnp.float32),
              jax.ShapeDtypeStruct((N,), jnp.float32)],
    mesh=plsc.VectorSubcoreMesh(core_axis_name="c", subcore_axis_name="s",
                                num_cores=NC, num_subcores=NS),
    compiler_params=pltpu.CompilerParams(needs_layout_passes=False,
                                         use_tc_tiling_on_sc=False),
    scratch_types=[
        pltpu.VMEM_SHARED((N, HH), jnp.float32),
        pltpu.VMEM_SHARED((N,), jnp.float32),
        pltpu.VMEM_SHARED((N,), jnp.float32),
        pltpu.VMEM_SHARED((N,), jnp.float32),
        pltpu.VMEM((SB * K,), jnp.int32),
        pltpu.VMEM((SB * K,), jnp.int32),
        pltpu.VMEM((2, K), jnp.int32),
        pltpu.VMEM((2, K), jnp.int32),
        pltpu.VMEM((2, K), jnp.int32),
        pltpu.VMEM((2, K), jnp.float32),
        pltpu.VMEM((2, K), jnp.float32),
        pltpu.VMEM((2, K), jnp.float32),
        pltpu.VMEM((2, K, HH), jnp.float32),
        pltpu.VMEM((RCNT,), jnp.float32),
        pltpu.VMEM((16,), jnp.float32),
        pltpu.SemaphoreType.DMA((2,)),
        pltpu.SemaphoreType.DMA((2,)),
        pltpu.SemaphoreType.DMA((2,)),
        pltpu.SemaphoreType.DMA((2,)),
        pltpu.SemaphoreType.DMA((2,)),
    ],
  )


def _pool_body(o1_ref, o2_ref, o3_ref, alo_ref, ahi_ref, den_ref, b_ref,
               bat_ref, out_ref):
    i = pl.program_id(0)
    acc = jnp.concatenate([alo_ref[...], ahi_ref[...]], axis=1)
    den = jnp.maximum(den_ref[...], 1e-30)
    out4 = jnp.maximum(acc / den + b_ref[...], 0.0)
    feats = jnp.concatenate([o1_ref[...], o2_ref[...], o3_ref[...], out4],
                            axis=1)
    bids = bat_ref[...]
    gids = lax.broadcasted_iota(jnp.int32, (G, BR), 0)
    onehot = (gids == bids[:, 0][None, :]).astype(jnp.float32)
    contrib = jnp.dot(onehot, feats, preferred_element_type=jnp.float32)

    @pl.when(i == 0)
    def _():
        out_ref[...] = jnp.zeros_like(out_ref)
    out_ref[...] += contrib


def _tc_pool(out1, out2, out3, acc4, den4, b4, batch):
    rows = lambda w: pl.BlockSpec((BR, w), lambda i: (i, 0))
    return pl.pallas_call(
        _pool_body,
        grid=(NBLK,),
        in_specs=[rows(H), rows(H), rows(H),
                  pl.BlockSpec((BR, HH), lambda i: (i, 0)),
                  pl.BlockSpec((BR, HH), lambda i: (NBLK + i, 0)),
                  rows(1),
                  pl.BlockSpec((1, H), lambda i: (0, 0)),
                  rows(1)],
        out_specs=pl.BlockSpec((G, 4 * H), lambda i: (0, 0)),
        out_shape=jax.ShapeDtypeStruct((G, 4 * H), jnp.float32),
    )(out1, out2, out3, acc4, acc4, den4, b4, batch)


def _mbound(mx):
    b = jnp.max(mx[:, 0, 0]) + jnp.max(mx[:, 0, 1])
    m = jnp.where(b >= 0.0, b, 0.2 * b)
    return jnp.full((16,), m, jnp.float32)


def kernel(x, edge_index, batch,
           W1, as1, ad1, b1,
           W2, as2, ad2, b2,
           W3, as3, ad3, b3,
           W4, as4, ad4, b4):
    loop = jnp.arange(N, dtype=edge_index.dtype)
    pad = jnp.zeros((EPAD - ETOT,), edge_index.dtype)
    srcp = jnp.concatenate([edge_index[0], loop, pad]).astype(jnp.int32)
    dstp = jnp.concatenate([edge_index[1], loop, pad]).astype(jnp.int32)
    batc = batch.astype(jnp.int32).reshape(N, 1)

    r2 = lambda v: v.reshape(1, H)
    hsplit = lambda h: h.reshape(2 * N, HH)

    h1, es1, ed1, mx1 = _tc_prep(True, False, 3,
                                 (x, W1, r2(as1), r2(ad1)))
    acc1, den1 = _get_sc_edge()(hsplit(h1), es1.reshape(N), ed1.reshape(N),
                                srcp, dstp, _mbound(mx1))

    out1, h2, es2, ed2, mx2 = _tc_prep(False, False, H,
                                       (acc1, acc1,
                                        den1.reshape(N, 1), r2(b1),
                                        W2, r2(as2), r2(ad2)))
    acc2, den2 = _get_sc_edge()(hsplit(h2), es2.reshape(N), ed2.reshape(N),
                                srcp, dstp, _mbound(mx2))

    out2, h3, es3, ed3, mx3 = _tc_prep(False, False, H,
                                       (acc2, acc2,
                                        den2.reshape(N, 1), r2(b2),
                                        W3, r2(as3), r2(ad3)))
    acc3, den3 = _get_sc_edge()(hsplit(h3), es3.reshape(N), ed3.reshape(N),
                                srcp, dstp, _mbound(mx3))

    out3, h4, es4, ed4, mx4 = _tc_prep(False, True, H,
                                       (acc3, acc3,
                                        den3.reshape(N, 1), r2(b3), out1,
                                        W4, r2(as4), r2(ad4)))
    acc4, den4 = _get_sc_edge()(hsplit(h4), es4.reshape(N), ed4.reshape(N),
                                srcp, dstp, _mbound(mx4))

    return _tc_pool(out1, out2, out3, acc4,
                    den4.reshape(N, 1), r2(b4), batc)
